# baseline (speedup 1.0000x reference)
import functools

import jax
import jax.numpy as jnp
from jax import lax
from jax.experimental import pallas as pl
from jax.experimental.pallas import tpu as pltpu
from jax.experimental.pallas import tpu_sc as plsc

_N = 83968
_E = 2686976
_B = 1024
_NN = 82
_SEQ = 20
_EMB = 20
_HID = 300
_NCLS = 22

_EMBP = 24
_NC = 2
_NS = 16
_NW = _NC * _NS
_EPW = _E // _NW
_K = 128
_ITERS = _EPW // _K
_RPS = _N // _NS
_ZW = 1312

_NSET = 4
_QIT = _ITERS // _NSET
_mesh = plsc.VectorSubcoreMesh(core_axis_name="c", subcore_axis_name="s")


_DCB = 8
_DCH = _ITERS // _DCB


def _sc_degree(dst2d):
    @functools.partial(
        pl.kernel,
        mesh=_mesh,
        out_type=jax.ShapeDtypeStruct((2 * _N,), jnp.float32),
        scratch_types=[
            [pltpu.VMEM((_DCB, _K), jnp.int32)] * 2,
            pltpu.VMEM((_K,), jnp.float32),
            pltpu.VMEM((_ZW,), jnp.float32),
            pltpu.VMEM_SHARED((_N,), jnp.float32),
            [pltpu.SemaphoreType.DMA] * 2,
            [pltpu.SemaphoreType.DMA] * 2,
        ],
    )
    def k(dst_hbm, out_hbm, db, ones_v, zer_v, deg_sh, isem, ssem):
        c = lax.axis_index("c")
        s = lax.axis_index("s")
        wid = s * _NC + c
        for j in range(_ZW // 16):
            zer_v[pl.ds(j * 16, 16)] = jnp.zeros((16,), jnp.float32)
        for j in range(_K // 16):
            ones_v[pl.ds(j * 16, 16)] = jnp.ones((16,), jnp.float32)
        row0 = pl.multiple_of(s * _RPS, 8)
        for j in range(_RPS // _ZW):
            pltpu.sync_copy(zer_v, deg_sh.at[pl.ds(row0 + j * _ZW, _ZW)])
        plsc.subcore_barrier()
        r0 = wid * _ITERS

        def issue_chunk(cc, p):
            b = pl.multiple_of(r0 + cc * _DCB, 8)
            pltpu.async_copy(dst_hbm.at[pl.ds(b, _DCB)], db[p], isem[p])

        def wait_chunk(p):
            pltpu.make_async_copy(
                dst_hbm.at[pl.ds(0, _DCB)], db[p], isem[p]).wait()

        def wait_scatter(h):
            pltpu.make_async_copy(
                dst_hbm.at[0], db[0].at[0], ssem[h]).wait()

        issue_chunk(0, 0)
        issue_chunk(1, 1)

        def do_chunk(cc, p, may_issue):
            wait_chunk(p)
            for j in range(_DCB):
                if j >= 2:
                    wait_scatter(j % 2)
                pltpu.async_copy(ones_v, deg_sh.at[db[p].at[j]],
                                 ssem[j % 2], add=True)
            wait_scatter(0)
            wait_scatter(1)

            @pl.when(may_issue)
            def _():
                issue_chunk(cc + 2, p)

        def body(q, carry):
            do_chunk(2 * q, 0, q * 2 + 2 < _DCH)
            do_chunk(2 * q + 1, 1, q * 2 + 3 < _DCH)
            return carry

        lax.fori_loop(0, _DCH // 2, body, 0)
        plsc.subcore_barrier()
        dump0 = pl.multiple_of(c * _N + row0, 8)
        pltpu.sync_copy(deg_sh.at[pl.ds(row0, _RPS)],
                        out_hbm.at[pl.ds(dump0, _RPS)])

    return k(dst2d)


_KH = _K // 2


def _sc_scatter(g, eb, zrows):
    @functools.partial(
        pl.kernel,
        mesh=_mesh,
        compiler_params=pltpu.CompilerParams(use_tc_tiling_on_sc=False),
        out_type=jax.ShapeDtypeStruct((2 * _N, _EMBP), jnp.float32),
        scratch_types=[
            [pltpu.VMEM((4, _KH), jnp.int32)] * _NSET,
            [pltpu.VMEM((_KH, _EMBP), jnp.float32)] * 2,
            pltpu.VMEM_SHARED((_N, _EMBP), jnp.float32),
            [pltpu.SemaphoreType.DMA] * _NSET,
            [pltpu.SemaphoreType.DMA] * 2,
            [pltpu.SemaphoreType.DMA] * 2,
        ],
    )
    def k(g_hbm, eb_hbm, z_hbm, out_hbm,
          eb, rows, acc_sh, isem, gsem, ssem):
        c = lax.axis_index("c")
        s = lax.axis_index("s")
        wid = s * _NC + c
        row0 = pl.multiple_of(s * _RPS, 8)
        pltpu.sync_copy(z_hbm.at[pl.ds(row0, _RPS)],
                        acc_sh.at[pl.ds(row0, _RPS)])
        plsc.subcore_barrier()
        b0 = wid * _ITERS

        def issue_idx(i, p):
            b = pl.multiple_of((b0 + i) * 4, 4)
            pltpu.async_copy(eb_hbm.at[pl.ds(b, 4)], eb[p], isem[p])

        def wait_idx(p):
            pltpu.make_async_copy(
                eb_hbm.at[pl.ds(0, 4)], eb[p], isem[p]).wait()

        def wait_scatter(h):
            pltpu.make_async_copy(
                g_hbm.at[pl.ds(0, _KH)], rows[h], ssem[h]).wait()

        def wait_gather(h):
            pltpu.make_async_copy(
                g_hbm.at[pl.ds(0, _KH)], rows[h], gsem[h]).wait()

        issue_idx(0, 0)
        issue_idx(1, 1)

        def body(q, carry):
            for r in range(_NSET):
                i = q * _NSET + r

                @pl.when(i >= 1)
                def _():
                    wait_scatter(0)

                wait_idx(r)

                @pl.when(i + 2 < _ITERS)
                def _():
                    issue_idx(i + 2, (r + 2) % _NSET)

                pltpu.async_copy(
                    g_hbm.at[eb[r].at[0]], rows[0], gsem[0])

                @pl.when(i >= 1)
                def _():
                    wait_scatter(1)

                pltpu.async_copy(
                    g_hbm.at[eb[r].at[1]], rows[1], gsem[1])
                wait_gather(0)
                pltpu.async_copy(rows[0], acc_sh.at[eb[r].at[2]], ssem[0],
                                 add=True)
                wait_gather(1)
                pltpu.async_copy(rows[1], acc_sh.at[eb[r].at[3]], ssem[1],
                                 add=True)
            return carry

        lax.fori_loop(0, _QIT, body, 0)
        wait_scatter(0)
        wait_scatter(1)
        plsc.subcore_barrier()
        dump0 = pl.multiple_of(c * _N + row0, 8)
        pltpu.sync_copy(acc_sh.at[pl.ds(row0, _RPS)],
                        out_hbm.at[pl.ds(dump0, _RPS)])

    return k(g, eb, zrows)


_R = 5248


def _prep1_body(deg_ref, x_ref, w_ref, dinv_ref, g_ref):
    deg = deg_ref[0, :] + deg_ref[1, :] + 1.0
    dinv = lax.rsqrt(deg)
    dinv_ref[...] = dinv[:, None]
    h = jnp.dot(x_ref[...], w_ref[...], preferred_element_type=jnp.float32)
    g_ref[...] = jnp.concatenate(
        [h * dinv[:, None], jnp.zeros((_R, _EMBP - _EMB), jnp.float32)], axis=1)


def _tc_prep1(deg2, x, W1c):
    return pl.pallas_call(
        _prep1_body,
        grid=(_N // _R,),
        in_specs=[
            pl.BlockSpec((2, _R), lambda i: (0, i)),
            pl.BlockSpec((_R, _SEQ), lambda i: (i, 0)),
            pl.BlockSpec((_SEQ, _EMB), lambda i: (0, 0)),
        ],
        out_specs=[
            pl.BlockSpec((_R, 1), lambda i: (i, 0)),
            pl.BlockSpec((_R, _EMBP), lambda i: (i, 0)),
        ],
        out_shape=[
            jax.ShapeDtypeStruct((_N, 1), jnp.float32),
            jax.ShapeDtypeStruct((_N, _EMBP), jnp.float32),
        ],
    )(deg2, x, W1c)


def _mid_body(g_ref, accp_ref, dinv_ref, b1_ref, w2_ref, g2_ref):
    acc = accp_ref[0, :, :_EMB] + accp_ref[1, :, :_EMB]
    dinv = dinv_ref[...]
    z1 = jnp.maximum(dinv * (g_ref[:, :_EMB] + acc) + b1_ref[...], 0.0)
    g2 = jnp.dot(z1, w2_ref[...], preferred_element_type=jnp.float32) * dinv
    g2_ref[...] = jnp.concatenate(
        [g2, jnp.zeros((_R, _EMBP - _EMB), jnp.float32)], axis=1)


def _tc_mid(g1, accp, dinv, b1, W2c):
    return pl.pallas_call(
        _mid_body,
        grid=(_N // _R,),
        in_specs=[
            pl.BlockSpec((_R, _EMBP), lambda i: (i, 0)),
            pl.BlockSpec((2, _R, _EMBP), lambda i: (0, i, 0)),
            pl.BlockSpec((_R, 1), lambda i: (i, 0)),
            pl.BlockSpec((1, _EMB), lambda i: (0, 0)),
            pl.BlockSpec((_EMB, _EMB), lambda i: (0, 0)),
        ],
        out_specs=pl.BlockSpec((_R, _EMBP), lambda i: (i, 0)),
        out_shape=jax.ShapeDtypeStruct((_N, _EMBP), jnp.float32),
    )(g1, accp, dinv, b1, W2c)


def _fin_body(g2_ref, accp_ref, dinv_ref, b2_ref, z2_ref):
    acc = accp_ref[0, :, :_EMB] + accp_ref[1, :, :_EMB]
    z2_ref[...] = jnp.maximum(
        dinv_ref[...] * (g2_ref[:, :_EMB] + acc) + b2_ref[...], 0.0)


def _tc_fin(g2, accp, dinv, b2):
    return pl.pallas_call(
        _fin_body,
        grid=(_N // _R,),
        in_specs=[
            pl.BlockSpec((_R, _EMBP), lambda i: (i, 0)),
            pl.BlockSpec((2, _R, _EMBP), lambda i: (0, i, 0)),
            pl.BlockSpec((_R, 1), lambda i: (i, 0)),
            pl.BlockSpec((1, _EMB), lambda i: (0, 0)),
        ],
        out_specs=pl.BlockSpec((_R, _EMB), lambda i: (i, 0)),
        out_shape=jax.ShapeDtypeStruct((_N, _EMB), jnp.float32),
    )(g2, accp, dinv, b2)


_GB = 256


def _head_body(lat_ref, wfc_ref, bfc_ref, wout_ref, bout_ref, o_ref):
    h = jnp.maximum(
        jnp.dot(lat_ref[...], wfc_ref[...],
                preferred_element_type=jnp.float32) + bfc_ref[...], 0.0)
    o_ref[...] = jnp.dot(h, wout_ref[...],
                         preferred_element_type=jnp.float32) + bout_ref[...]


def _tc_head(lat, Wfc, bfc, Wout, bout):
    return pl.pallas_call(
        _head_body,
        grid=(_B // _GB,),
        in_specs=[
            pl.BlockSpec((_GB, _NN * _EMB), lambda i: (i, 0)),
            pl.BlockSpec((_NN * _EMB, _HID), lambda i: (0, 0)),
            pl.BlockSpec((1, _HID), lambda i: (0, 0)),
            pl.BlockSpec((_HID, _NCLS), lambda i: (0, 0)),
            pl.BlockSpec((1, _NCLS), lambda i: (0, 0)),
        ],
        out_specs=pl.BlockSpec((_GB, _NCLS), lambda i: (i, 0)),
        out_shape=jax.ShapeDtypeStruct((_B, _NCLS), jnp.float32),
    )(lat, Wfc, bfc, Wout, bout)


def kernel(x, edge_index, batch_index, W1c, b1c, W2c, b2c, Wfc, bfc, Wout, bout):
    src = edge_index[0]
    dst = edge_index[1]
    eb = jnp.concatenate(
        [src.reshape(_E // _K, 2, _KH), dst.reshape(_E // _K, 2, _KH)],
        axis=1).reshape((_E // _K) * 4, _KH)
    dst2 = dst.reshape(_E // _K, _K)
    zrows = jnp.zeros((_N, _EMBP), jnp.float32)

    degp = _sc_degree(dst2)
    deg2 = degp.reshape(2, _N)
    dinv, g1 = _tc_prep1(deg2, x, W1c)

    acc1p = _sc_scatter(g1, eb, zrows).reshape(2, _N, _EMBP)
    g2 = _tc_mid(g1, acc1p, dinv, b1c.reshape(1, _EMB), W2c)

    acc2p = _sc_scatter(g2, eb, zrows).reshape(2, _N, _EMBP)
    z2 = _tc_fin(g2, acc2p, dinv, b2c.reshape(1, _EMB))

    lat = z2.reshape(_B, _NN * _EMB)
    return _tc_head(lat, Wfc, bfc.reshape(1, _HID), Wout, bout.reshape(1, _NCLS))


# speedup vs baseline: 1.1432x; 1.1432x over previous
import functools

import jax
import jax.numpy as jnp
from jax import lax
from jax.experimental import pallas as pl
from jax.experimental.pallas import tpu as pltpu
from jax.experimental.pallas import tpu_sc as plsc

_N = 83968
_E = 2686976
_B = 1024
_NN = 82
_SEQ = 20
_EMB = 20
_HID = 300
_NCLS = 22

_EMBP = 24
_NC = 2
_NS = 16
_NW = _NC * _NS
_EPW = _E // _NW
_K = 128
_ITERS = _EPW // _K
_RPS = _N // _NS
_ZW = 1312

_NSET = 4
_QIT = _ITERS // _NSET
_mesh = plsc.VectorSubcoreMesh(core_axis_name="c", subcore_axis_name="s")


_DCB = 8
_DCH = _ITERS // _DCB


def _sc_degree(dst2d):
    @functools.partial(
        pl.kernel,
        mesh=_mesh,
        out_type=jax.ShapeDtypeStruct((2 * _N,), jnp.float32),
        scratch_types=[
            [pltpu.VMEM((_DCB, _K), jnp.int32)] * 2,
            pltpu.VMEM((_K,), jnp.float32),
            pltpu.VMEM((_ZW,), jnp.float32),
            pltpu.VMEM_SHARED((_N,), jnp.float32),
            [pltpu.SemaphoreType.DMA] * 2,
            [pltpu.SemaphoreType.DMA] * 2,
        ],
    )
    def k(dst_hbm, out_hbm, db, ones_v, zer_v, deg_sh, isem, ssem):
        c = lax.axis_index("c")
        s = lax.axis_index("s")
        wid = s * _NC + c
        for j in range(_ZW // 16):
            zer_v[pl.ds(j * 16, 16)] = jnp.zeros((16,), jnp.float32)
        for j in range(_K // 16):
            ones_v[pl.ds(j * 16, 16)] = jnp.ones((16,), jnp.float32)
        row0 = pl.multiple_of(s * _RPS, 8)
        for j in range(_RPS // _ZW):
            pltpu.sync_copy(zer_v, deg_sh.at[pl.ds(row0 + j * _ZW, _ZW)])
        plsc.subcore_barrier()
        r0 = wid * _ITERS

        def issue_chunk(cc, p):
            b = pl.multiple_of(r0 + cc * _DCB, 8)
            pltpu.async_copy(dst_hbm.at[pl.ds(b, _DCB)], db[p], isem[p])

        def wait_chunk(p):
            pltpu.make_async_copy(
                dst_hbm.at[pl.ds(0, _DCB)], db[p], isem[p]).wait()

        def wait_scatter(h):
            pltpu.make_async_copy(
                dst_hbm.at[0], db[0].at[0], ssem[h]).wait()

        issue_chunk(0, 0)
        issue_chunk(1, 1)

        def do_chunk(cc, p, may_issue):
            wait_chunk(p)
            for j in range(_DCB):
                if j >= 2:
                    wait_scatter(j % 2)
                pltpu.async_copy(ones_v, deg_sh.at[db[p].at[j]],
                                 ssem[j % 2], add=True)
            wait_scatter(0)
            wait_scatter(1)

            @pl.when(may_issue)
            def _():
                issue_chunk(cc + 2, p)

        def body(q, carry):
            do_chunk(2 * q, 0, q * 2 + 2 < _DCH)
            do_chunk(2 * q + 1, 1, q * 2 + 3 < _DCH)
            return carry

        lax.fori_loop(0, _DCH // 2, body, 0)
        plsc.subcore_barrier()
        dump0 = pl.multiple_of(c * _N + row0, 8)
        pltpu.sync_copy(deg_sh.at[pl.ds(row0, _RPS)],
                        out_hbm.at[pl.ds(dump0, _RPS)])

    return k(dst2d)


_KH = _K // 2


def _sc_scatter(g, src, dst2h, zrows):
    @functools.partial(
        pl.kernel,
        mesh=_mesh,
        compiler_params=pltpu.CompilerParams(use_tc_tiling_on_sc=False),
        out_type=jax.ShapeDtypeStruct((2 * _N, _EMBP), jnp.float32),
        scratch_types=[
            [pltpu.VMEM((_K,), jnp.int32)] * _NSET,
            [pltpu.VMEM((2, _KH), jnp.int32)] * _NSET,
            [pltpu.VMEM((_KH, _EMBP), jnp.float32)] * 2,
            pltpu.VMEM_SHARED((_N, _EMBP), jnp.float32),
            [pltpu.SemaphoreType.DMA] * _NSET,
            [pltpu.SemaphoreType.DMA] * 2,
            [pltpu.SemaphoreType.DMA] * 2,
        ],
    )
    def k(g_hbm, src_hbm, dst_hbm, z_hbm, out_hbm,
          si, di, rows, acc_sh, isem, gsem, ssem):
        c = lax.axis_index("c")
        s = lax.axis_index("s")
        wid = s * _NC + c
        row0 = pl.multiple_of(s * _RPS, 8)
        pltpu.sync_copy(z_hbm.at[pl.ds(row0, _RPS)],
                        acc_sh.at[pl.ds(row0, _RPS)])
        plsc.subcore_barrier()
        e0 = wid * _EPW

        def issue_idx(i, p):
            b = pl.multiple_of(e0 + i * _K, 8)
            pltpu.async_copy(src_hbm.at[pl.ds(b, _K)], si[p], isem[p])
            pltpu.async_copy(
                dst_hbm.at[pl.ds((e0 // _KH) + i * 2, 2)], di[p], isem[p])

        def wait_idx(p):
            pltpu.make_async_copy(
                src_hbm.at[pl.ds(0, _K)], si[p], isem[p]).wait()
            pltpu.make_async_copy(
                dst_hbm.at[pl.ds(0, 2)], di[p], isem[p]).wait()

        def wait_scatter(h):
            pltpu.make_async_copy(
                g_hbm.at[pl.ds(0, _KH)], rows[h], ssem[h]).wait()

        def wait_gather(h):
            pltpu.make_async_copy(
                g_hbm.at[pl.ds(0, _KH)], rows[h], gsem[h]).wait()

        issue_idx(0, 0)
        issue_idx(1, 1)

        def body(q, carry):
            for r in range(_NSET):
                i = q * _NSET + r

                @pl.when(i >= 1)
                def _():
                    wait_scatter(0)

                wait_idx(r)

                @pl.when(i + 2 < _ITERS)
                def _():
                    issue_idx(i + 2, (r + 2) % _NSET)

                pltpu.async_copy(
                    g_hbm.at[si[r].at[pl.ds(0, _KH)]], rows[0], gsem[0])

                @pl.when(i >= 1)
                def _():
                    wait_scatter(1)

                pltpu.async_copy(
                    g_hbm.at[si[r].at[pl.ds(_KH, _KH)]], rows[1], gsem[1])
                wait_gather(0)
                pltpu.async_copy(rows[0], acc_sh.at[di[r].at[0]], ssem[0],
                                 add=True)
                wait_gather(1)
                pltpu.async_copy(rows[1], acc_sh.at[di[r].at[1]], ssem[1],
                                 add=True)
            return carry

        lax.fori_loop(0, _QIT, body, 0)
        wait_scatter(0)
        wait_scatter(1)
        plsc.subcore_barrier()
        dump0 = pl.multiple_of(c * _N + row0, 8)
        pltpu.sync_copy(acc_sh.at[pl.ds(row0, _RPS)],
                        out_hbm.at[pl.ds(dump0, _RPS)])

    return k(g, src, dst2h, zrows)


_R = 5248


def _prep1_body(deg_ref, x_ref, w_ref, dinv_ref, g_ref):
    deg = deg_ref[0, :] + deg_ref[1, :] + 1.0
    dinv = lax.rsqrt(deg)
    dinv_ref[...] = dinv[:, None]
    h = jnp.dot(x_ref[...], w_ref[...], preferred_element_type=jnp.float32)
    g_ref[...] = jnp.concatenate(
        [h * dinv[:, None], jnp.zeros((_R, _EMBP - _EMB), jnp.float32)], axis=1)


def _tc_prep1(deg2, x, W1c):
    return pl.pallas_call(
        _prep1_body,
        grid=(_N // _R,),
        in_specs=[
            pl.BlockSpec((2, _R), lambda i: (0, i)),
            pl.BlockSpec((_R, _SEQ), lambda i: (i, 0)),
            pl.BlockSpec((_SEQ, _EMB), lambda i: (0, 0)),
        ],
        out_specs=[
            pl.BlockSpec((_R, 1), lambda i: (i, 0)),
            pl.BlockSpec((_R, _EMBP), lambda i: (i, 0)),
        ],
        out_shape=[
            jax.ShapeDtypeStruct((_N, 1), jnp.float32),
            jax.ShapeDtypeStruct((_N, _EMBP), jnp.float32),
        ],
    )(deg2, x, W1c)


def _mid_body(g_ref, accp_ref, dinv_ref, b1_ref, w2_ref, g2_ref):
    acc = accp_ref[0, :, :_EMB] + accp_ref[1, :, :_EMB]
    dinv = dinv_ref[...]
    z1 = jnp.maximum(dinv * (g_ref[:, :_EMB] + acc) + b1_ref[...], 0.0)
    g2 = jnp.dot(z1, w2_ref[...], preferred_element_type=jnp.float32) * dinv
    g2_ref[...] = jnp.concatenate(
        [g2, jnp.zeros((_R, _EMBP - _EMB), jnp.float32)], axis=1)


def _tc_mid(g1, accp, dinv, b1, W2c):
    return pl.pallas_call(
        _mid_body,
        grid=(_N // _R,),
        in_specs=[
            pl.BlockSpec((_R, _EMBP), lambda i: (i, 0)),
            pl.BlockSpec((2, _R, _EMBP), lambda i: (0, i, 0)),
            pl.BlockSpec((_R, 1), lambda i: (i, 0)),
            pl.BlockSpec((1, _EMB), lambda i: (0, 0)),
            pl.BlockSpec((_EMB, _EMB), lambda i: (0, 0)),
        ],
        out_specs=pl.BlockSpec((_R, _EMBP), lambda i: (i, 0)),
        out_shape=jax.ShapeDtypeStruct((_N, _EMBP), jnp.float32),
    )(g1, accp, dinv, b1, W2c)


def _fin_body(g2_ref, accp_ref, dinv_ref, b2_ref, z2_ref):
    acc = accp_ref[0, :, :_EMB] + accp_ref[1, :, :_EMB]
    z2_ref[...] = jnp.maximum(
        dinv_ref[...] * (g2_ref[:, :_EMB] + acc) + b2_ref[...], 0.0)


def _tc_fin(g2, accp, dinv, b2):
    return pl.pallas_call(
        _fin_body,
        grid=(_N // _R,),
        in_specs=[
            pl.BlockSpec((_R, _EMBP), lambda i: (i, 0)),
            pl.BlockSpec((2, _R, _EMBP), lambda i: (0, i, 0)),
            pl.BlockSpec((_R, 1), lambda i: (i, 0)),
            pl.BlockSpec((1, _EMB), lambda i: (0, 0)),
        ],
        out_specs=pl.BlockSpec((_R, _EMB), lambda i: (i, 0)),
        out_shape=jax.ShapeDtypeStruct((_N, _EMB), jnp.float32),
    )(g2, accp, dinv, b2)


_GB = 256


def _head_body(lat_ref, wfc_ref, bfc_ref, wout_ref, bout_ref, o_ref):
    h = jnp.maximum(
        jnp.dot(lat_ref[...], wfc_ref[...],
                preferred_element_type=jnp.float32) + bfc_ref[...], 0.0)
    o_ref[...] = jnp.dot(h, wout_ref[...],
                         preferred_element_type=jnp.float32) + bout_ref[...]


def _tc_head(lat, Wfc, bfc, Wout, bout):
    return pl.pallas_call(
        _head_body,
        grid=(_B // _GB,),
        in_specs=[
            pl.BlockSpec((_GB, _NN * _EMB), lambda i: (i, 0)),
            pl.BlockSpec((_NN * _EMB, _HID), lambda i: (0, 0)),
            pl.BlockSpec((1, _HID), lambda i: (0, 0)),
            pl.BlockSpec((_HID, _NCLS), lambda i: (0, 0)),
            pl.BlockSpec((1, _NCLS), lambda i: (0, 0)),
        ],
        out_specs=pl.BlockSpec((_GB, _NCLS), lambda i: (i, 0)),
        out_shape=jax.ShapeDtypeStruct((_B, _NCLS), jnp.float32),
    )(lat, Wfc, bfc, Wout, bout)


def kernel(x, edge_index, batch_index, W1c, b1c, W2c, b2c, Wfc, bfc, Wout, bout):
    src = edge_index[0]
    dst = edge_index[1]
    dst2h = dst.reshape(_E // _KH, _KH)
    dst2 = dst.reshape(_E // _K, _K)
    zrows = jnp.zeros((_N, _EMBP), jnp.float32)

    degp = _sc_degree(dst2)
    deg2 = degp.reshape(2, _N)
    dinv, g1 = _tc_prep1(deg2, x, W1c)

    acc1p = _sc_scatter(g1, src, dst2h, zrows).reshape(2, _N, _EMBP)
    g2 = _tc_mid(g1, acc1p, dinv, b1c.reshape(1, _EMB), W2c)

    acc2p = _sc_scatter(g2, src, dst2h, zrows).reshape(2, _N, _EMBP)
    z2 = _tc_fin(g2, acc2p, dinv, b2c.reshape(1, _EMB))

    lat = z2.reshape(_B, _NN * _EMB)
    return _tc_head(lat, Wfc, bfc.reshape(1, _HID), Wout, bout.reshape(1, _NCLS))


# speedup vs baseline: 1.1639x; 1.0182x over previous
import functools

import jax
import jax.numpy as jnp
from jax import lax
from jax.experimental import pallas as pl
from jax.experimental.pallas import tpu as pltpu
from jax.experimental.pallas import tpu_sc as plsc

_N = 83968
_E = 2686976
_B = 1024
_NN = 82
_SEQ = 20
_EMB = 20
_HID = 300
_NCLS = 22

_EMBP = 24
_NC = 2
_NS = 16
_NW = _NC * _NS
_EPW = _E // _NW
_K = 128
_ITERS = _EPW // _K
_RPS = _N // _NS
_ZW = 1312

_NSET = 4
_QIT = _ITERS // _NSET
_mesh = plsc.VectorSubcoreMesh(core_axis_name="c", subcore_axis_name="s")


_DCB = 8
_DCH = _ITERS // _DCB


def _sc_degree(dst2d):
    @functools.partial(
        pl.kernel,
        mesh=_mesh,
        out_type=jax.ShapeDtypeStruct((2 * _N,), jnp.float32),
        scratch_types=[
            [pltpu.VMEM((_DCB, _K), jnp.int32)] * 2,
            pltpu.VMEM((_K,), jnp.float32),
            pltpu.VMEM((_ZW,), jnp.float32),
            pltpu.VMEM_SHARED((_N,), jnp.float32),
            [pltpu.SemaphoreType.DMA] * 2,
            [pltpu.SemaphoreType.DMA] * 2,
        ],
    )
    def k(dst_hbm, out_hbm, db, ones_v, zer_v, deg_sh, isem, ssem):
        c = lax.axis_index("c")
        s = lax.axis_index("s")
        wid = s * _NC + c
        for j in range(_ZW // 16):
            zer_v[pl.ds(j * 16, 16)] = jnp.zeros((16,), jnp.float32)
        for j in range(_K // 16):
            ones_v[pl.ds(j * 16, 16)] = jnp.ones((16,), jnp.float32)
        row0 = pl.multiple_of(s * _RPS, 8)
        for j in range(_RPS // _ZW):
            pltpu.sync_copy(zer_v, deg_sh.at[pl.ds(row0 + j * _ZW, _ZW)])
        plsc.subcore_barrier()
        r0 = wid * _ITERS

        def issue_chunk(cc, p):
            b = pl.multiple_of(r0 + cc * _DCB, 8)
            pltpu.async_copy(dst_hbm.at[pl.ds(b, _DCB)], db[p], isem[p])

        def wait_chunk(p):
            pltpu.make_async_copy(
                dst_hbm.at[pl.ds(0, _DCB)], db[p], isem[p]).wait()

        def wait_scatter(h):
            pltpu.make_async_copy(
                dst_hbm.at[0], db[0].at[0], ssem[h]).wait()

        issue_chunk(0, 0)
        issue_chunk(1, 1)

        def do_chunk(cc, p, may_issue):
            wait_chunk(p)
            for j in range(_DCB):
                if j >= 2:
                    wait_scatter(j % 2)
                pltpu.async_copy(ones_v, deg_sh.at[db[p].at[j]],
                                 ssem[j % 2], add=True)
            wait_scatter(0)
            wait_scatter(1)

            @pl.when(may_issue)
            def _():
                issue_chunk(cc + 2, p)

        def body(q, carry):
            do_chunk(2 * q, 0, q * 2 + 2 < _DCH)
            do_chunk(2 * q + 1, 1, q * 2 + 3 < _DCH)
            return carry

        lax.fori_loop(0, _DCH // 2, body, 0)
        plsc.subcore_barrier()
        dump0 = pl.multiple_of(c * _N + row0, 8)
        pltpu.sync_copy(deg_sh.at[pl.ds(row0, _RPS)],
                        out_hbm.at[pl.ds(dump0, _RPS)])

    return k(dst2d)


_KH = _K // 2


def _sc_scatter(g, src, dst2h, zrows):
    @functools.partial(
        pl.kernel,
        mesh=_mesh,
        compiler_params=pltpu.CompilerParams(use_tc_tiling_on_sc=False),
        out_type=jax.ShapeDtypeStruct((2 * _N, _EMBP), jnp.float32),
        scratch_types=[
            [pltpu.VMEM((4, _K), jnp.int32)] * 2,
            [pltpu.VMEM((2, _KH), jnp.int32)] * _NSET,
            [pltpu.VMEM((_KH, _EMBP), jnp.float32)] * 2,
            pltpu.VMEM_SHARED((_N, _EMBP), jnp.float32),
            [pltpu.SemaphoreType.DMA] * 2,
            [pltpu.SemaphoreType.DMA] * _NSET,
            [pltpu.SemaphoreType.DMA] * 2,
            [pltpu.SemaphoreType.DMA] * 2,
        ],
    )
    def k(g_hbm, src_hbm, dst_hbm, z_hbm, out_hbm,
          si, di, rows, acc_sh, csem, dsem, gsem, ssem):
        c = lax.axis_index("c")
        s = lax.axis_index("s")
        wid = s * _NC + c
        row0 = pl.multiple_of(s * _RPS, 8)

        @pl.when(c == 0)
        def _():
            pltpu.sync_copy(g_hbm.at[pl.ds(row0, _RPS)],
                            acc_sh.at[pl.ds(row0, _RPS)])

        @pl.when(c != 0)
        def _():
            pltpu.sync_copy(z_hbm.at[pl.ds(row0, _RPS)],
                            acc_sh.at[pl.ds(row0, _RPS)])

        plsc.subcore_barrier()
        e0 = wid * _EPW
        b0 = wid * _ITERS

        def issue_chunk(cc, p):
            b = pl.multiple_of(b0 + cc * 4, 4)
            pltpu.async_copy(src_hbm.at[pl.ds(b, 4)], si[p], csem[p])

        def wait_chunk(p):
            pltpu.make_async_copy(
                src_hbm.at[pl.ds(0, 4)], si[p], csem[p]).wait()

        def issue_di(i, p):
            pltpu.async_copy(
                dst_hbm.at[pl.ds((e0 // _KH) + i * 2, 2)], di[p], dsem[p])

        def wait_di(p):
            pltpu.make_async_copy(
                dst_hbm.at[pl.ds(0, 2)], di[p], dsem[p]).wait()

        def wait_scatter(h):
            pltpu.make_async_copy(
                g_hbm.at[pl.ds(0, _KH)], rows[h], ssem[h]).wait()

        def wait_gather(h):
            pltpu.make_async_copy(
                g_hbm.at[pl.ds(0, _KH)], rows[h], gsem[h]).wait()

        issue_chunk(0, 0)
        issue_chunk(1, 1)
        issue_di(0, 0)
        issue_di(1, 1)

        def body(u, carry):
            for r in range(8):
                i = u * 8 + r
                sp = (r // 4) % 2

                if r == 0:
                    wait_chunk(0)
                if r == 4:
                    wait_chunk(1)

                    @pl.when(u < (_ITERS // 8) - 1)
                    def _():
                        issue_chunk(u * 2 + 2, 0)

                @pl.when(i >= 1)
                def _():
                    wait_scatter(0)

                wait_di(r % _NSET)

                @pl.when(i + 2 < _ITERS)
                def _():
                    issue_di(i + 2, (r + 2) % _NSET)

                pltpu.async_copy(
                    g_hbm.at[si[sp].at[r % 4, pl.ds(0, _KH)]],
                    rows[0], gsem[0])

                @pl.when(i >= 1)
                def _():
                    wait_scatter(1)

                pltpu.async_copy(
                    g_hbm.at[si[sp].at[r % 4, pl.ds(_KH, _KH)]],
                    rows[1], gsem[1])
                wait_gather(0)
                pltpu.async_copy(rows[0], acc_sh.at[di[r % _NSET].at[0]],
                                 ssem[0], add=True)
                wait_gather(1)
                pltpu.async_copy(rows[1], acc_sh.at[di[r % _NSET].at[1]],
                                 ssem[1], add=True)

            @pl.when(u < (_ITERS // 8) - 1)
            def _():
                issue_chunk(u * 2 + 3, 1)

            return carry

        lax.fori_loop(0, _ITERS // 8, body, 0)
        wait_scatter(0)
        wait_scatter(1)
        plsc.subcore_barrier()
        dump0 = pl.multiple_of(c * _N + row0, 8)
        pltpu.sync_copy(acc_sh.at[pl.ds(row0, _RPS)],
                        out_hbm.at[pl.ds(dump0, _RPS)])

    return k(g, src, dst2h, zrows)


_R = 5248


def _prep1_body(deg_ref, x_ref, w_ref, dinv_ref, g_ref):
    deg = deg_ref[0, :] + deg_ref[1, :] + 1.0
    dinv = lax.rsqrt(deg)
    dinv_ref[...] = dinv[:, None]
    h = jnp.dot(x_ref[...], w_ref[...], preferred_element_type=jnp.float32)
    g_ref[...] = jnp.concatenate(
        [h * dinv[:, None], jnp.zeros((_R, _EMBP - _EMB), jnp.float32)], axis=1)


def _tc_prep1(deg2, x, W1c):
    return pl.pallas_call(
        _prep1_body,
        grid=(_N // _R,),
        in_specs=[
            pl.BlockSpec((2, _R), lambda i: (0, i)),
            pl.BlockSpec((_R, _SEQ), lambda i: (i, 0)),
            pl.BlockSpec((_SEQ, _EMB), lambda i: (0, 0)),
        ],
        out_specs=[
            pl.BlockSpec((_R, 1), lambda i: (i, 0)),
            pl.BlockSpec((_R, _EMBP), lambda i: (i, 0)),
        ],
        out_shape=[
            jax.ShapeDtypeStruct((_N, 1), jnp.float32),
            jax.ShapeDtypeStruct((_N, _EMBP), jnp.float32),
        ],
    )(deg2, x, W1c)


def _mid_body(accp_ref, dinv_ref, b1_ref, w2_ref, g2_ref):
    acc = accp_ref[0, :, :_EMB] + accp_ref[1, :, :_EMB]
    dinv = dinv_ref[...]
    z1 = jnp.maximum(dinv * acc + b1_ref[...], 0.0)
    g2 = jnp.dot(z1, w2_ref[...], preferred_element_type=jnp.float32) * dinv
    g2_ref[...] = jnp.concatenate(
        [g2, jnp.zeros((_R, _EMBP - _EMB), jnp.float32)], axis=1)


def _tc_mid(accp, dinv, b1, W2c):
    return pl.pallas_call(
        _mid_body,
        grid=(_N // _R,),
        in_specs=[
            pl.BlockSpec((2, _R, _EMBP), lambda i: (0, i, 0)),
            pl.BlockSpec((_R, 1), lambda i: (i, 0)),
            pl.BlockSpec((1, _EMB), lambda i: (0, 0)),
            pl.BlockSpec((_EMB, _EMB), lambda i: (0, 0)),
        ],
        out_specs=pl.BlockSpec((_R, _EMBP), lambda i: (i, 0)),
        out_shape=jax.ShapeDtypeStruct((_N, _EMBP), jnp.float32),
    )(accp, dinv, b1, W2c)


def _fin_body(accp_ref, dinv_ref, b2_ref, z2_ref):
    acc = accp_ref[0, :, :_EMB] + accp_ref[1, :, :_EMB]
    z2_ref[...] = jnp.maximum(
        dinv_ref[...] * acc + b2_ref[...], 0.0)


def _tc_fin(accp, dinv, b2):
    return pl.pallas_call(
        _fin_body,
        grid=(_N // _R,),
        in_specs=[
            pl.BlockSpec((2, _R, _EMBP), lambda i: (0, i, 0)),
            pl.BlockSpec((_R, 1), lambda i: (i, 0)),
            pl.BlockSpec((1, _EMB), lambda i: (0, 0)),
        ],
        out_specs=pl.BlockSpec((_R, _EMB), lambda i: (i, 0)),
        out_shape=jax.ShapeDtypeStruct((_N, _EMB), jnp.float32),
    )(accp, dinv, b2)


_GB = 256


def _head_body(lat_ref, wfc_ref, bfc_ref, wout_ref, bout_ref, o_ref):
    h = jnp.maximum(
        jnp.dot(lat_ref[...], wfc_ref[...],
                preferred_element_type=jnp.float32) + bfc_ref[...], 0.0)
    o_ref[...] = jnp.dot(h, wout_ref[...],
                         preferred_element_type=jnp.float32) + bout_ref[...]


def _tc_head(lat, Wfc, bfc, Wout, bout):
    return pl.pallas_call(
        _head_body,
        grid=(_B // _GB,),
        in_specs=[
            pl.BlockSpec((_GB, _NN * _EMB), lambda i: (i, 0)),
            pl.BlockSpec((_NN * _EMB, _HID), lambda i: (0, 0)),
            pl.BlockSpec((1, _HID), lambda i: (0, 0)),
            pl.BlockSpec((_HID, _NCLS), lambda i: (0, 0)),
            pl.BlockSpec((1, _NCLS), lambda i: (0, 0)),
        ],
        out_specs=pl.BlockSpec((_GB, _NCLS), lambda i: (i, 0)),
        out_shape=jax.ShapeDtypeStruct((_B, _NCLS), jnp.float32),
    )(lat, Wfc, bfc, Wout, bout)


def kernel(x, edge_index, batch_index, W1c, b1c, W2c, b2c, Wfc, bfc, Wout, bout):
    src = edge_index[0]
    dst = edge_index[1]
    src4 = src.reshape(_E // _K, _K)
    dst2h = dst.reshape(_E // _KH, _KH)
    dst2 = dst.reshape(_E // _K, _K)
    zrows = jnp.zeros((_N, _EMBP), jnp.float32)

    degp = _sc_degree(dst2)
    deg2 = degp.reshape(2, _N)
    dinv, g1 = _tc_prep1(deg2, x, W1c)

    acc1p = _sc_scatter(g1, src4, dst2h, zrows).reshape(2, _N, _EMBP)
    g2 = _tc_mid(acc1p, dinv, b1c.reshape(1, _EMB), W2c)

    acc2p = _sc_scatter(g2, src4, dst2h, zrows).reshape(2, _N, _EMBP)
    z2 = _tc_fin(acc2p, dinv, b2c.reshape(1, _EMB))

    lat = z2.reshape(_B, _NN * _EMB)
    return _tc_head(lat, Wfc, bfc.reshape(1, _HID), Wout, bout.reshape(1, _NCLS))


# speedup vs baseline: 1.1664x; 1.0021x over previous
import functools

import jax
import jax.numpy as jnp
from jax import lax
from jax.experimental import pallas as pl
from jax.experimental.pallas import tpu as pltpu
from jax.experimental.pallas import tpu_sc as plsc

_N = 83968
_E = 2686976
_B = 1024
_NN = 82
_SEQ = 20
_EMB = 20
_HID = 300
_NCLS = 22

_EMBP = 24
_NC = 2
_NS = 16
_NW = _NC * _NS
_EPW = _E // _NW
_K = 128
_ITERS = _EPW // _K
_RPS = _N // _NS
_ZW = 1312

_NSET = 4
_mesh = plsc.VectorSubcoreMesh(core_axis_name="c", subcore_axis_name="s")


_DCB = 8
_DCH = _ITERS // _DCB


def _sc_degree(dst2d):
    @functools.partial(
        pl.kernel,
        mesh=_mesh,
        out_type=jax.ShapeDtypeStruct((2 * _N,), jnp.float32),
        scratch_types=[
            [pltpu.VMEM((_DCB, _K), jnp.int32)] * 2,
            pltpu.VMEM((_K,), jnp.float32),
            pltpu.VMEM((_ZW,), jnp.float32),
            pltpu.VMEM_SHARED((_N,), jnp.float32),
            [pltpu.SemaphoreType.DMA] * 2,
            [pltpu.SemaphoreType.DMA] * 2,
        ],
    )
    def k(dst_hbm, out_hbm, db, ones_v, zer_v, deg_sh, isem, ssem):
        c = lax.axis_index("c")
        s = lax.axis_index("s")
        wid = s * _NC + c
        for j in range(_ZW // 16):
            zer_v[pl.ds(j * 16, 16)] = jnp.zeros((16,), jnp.float32)
        for j in range(_K // 16):
            ones_v[pl.ds(j * 16, 16)] = jnp.ones((16,), jnp.float32)
        row0 = pl.multiple_of(s * _RPS, 8)
        for j in range(_RPS // _ZW):
            pltpu.sync_copy(zer_v, deg_sh.at[pl.ds(row0 + j * _ZW, _ZW)])
        plsc.subcore_barrier()
        r0 = wid * _ITERS

        def issue_chunk(cc, p):
            b = pl.multiple_of(r0 + cc * _DCB, 8)
            pltpu.async_copy(dst_hbm.at[pl.ds(b, _DCB)], db[p], isem[p])

        def wait_chunk(p):
            pltpu.make_async_copy(
                dst_hbm.at[pl.ds(0, _DCB)], db[p], isem[p]).wait()

        def wait_scatter(h):
            pltpu.make_async_copy(
                dst_hbm.at[0], db[0].at[0], ssem[h]).wait()

        issue_chunk(0, 0)
        issue_chunk(1, 1)

        def do_chunk(cc, p, may_issue):
            wait_chunk(p)
            for j in range(_DCB):
                if j >= 2:
                    wait_scatter(j % 2)
                pltpu.async_copy(ones_v, deg_sh.at[db[p].at[j]],
                                 ssem[j % 2], add=True)
            wait_scatter(0)
            wait_scatter(1)

            @pl.when(may_issue)
            def _():
                issue_chunk(cc + 2, p)

        def body(q, carry):
            do_chunk(2 * q, 0, q * 2 + 2 < _DCH)
            do_chunk(2 * q + 1, 1, q * 2 + 3 < _DCH)
            return carry

        lax.fori_loop(0, _DCH // 2, body, 0)
        plsc.subcore_barrier()
        dump0 = pl.multiple_of(c * _N + row0, 8)
        pltpu.sync_copy(deg_sh.at[pl.ds(row0, _RPS)],
                        out_hbm.at[pl.ds(dump0, _RPS)])

    return k(dst2d)


_KH = _K // 2


def _sc_scatter(g, src, dst2h, zrows):
    @functools.partial(
        pl.kernel,
        mesh=_mesh,
        compiler_params=pltpu.CompilerParams(use_tc_tiling_on_sc=False),
        out_type=jax.ShapeDtypeStruct((2 * _N, _EMBP), jnp.float32),
        scratch_types=[
            [pltpu.VMEM((4, _K), jnp.int32)] * 2,
            [pltpu.VMEM((2, _KH), jnp.int32)] * _NSET,
            [pltpu.VMEM((_KH, _EMBP), jnp.float32)] * 2,
            pltpu.VMEM_SHARED((_N, _EMBP), jnp.float32),
            [pltpu.SemaphoreType.DMA] * 2,
            [pltpu.SemaphoreType.DMA] * _NSET,
            [pltpu.SemaphoreType.DMA] * 2,
            [pltpu.SemaphoreType.DMA] * 2,
        ],
    )
    def k(g_hbm, src_hbm, dst_hbm, z_hbm, out_hbm,
          si, di, rows, acc_sh, csem, dsem, gsem, ssem):
        c = lax.axis_index("c")
        s = lax.axis_index("s")
        wid = s * _NC + c
        row0 = pl.multiple_of(s * _RPS, 8)

        @pl.when(c == 0)
        def _():
            pltpu.sync_copy(g_hbm.at[pl.ds(row0, _RPS)],
                            acc_sh.at[pl.ds(row0, _RPS)])

        @pl.when(c != 0)
        def _():
            pltpu.sync_copy(z_hbm.at[pl.ds(row0, _RPS)],
                            acc_sh.at[pl.ds(row0, _RPS)])

        plsc.subcore_barrier()
        e0 = wid * _EPW
        b0 = wid * _ITERS

        def issue_chunk(cc, p):
            b = pl.multiple_of(b0 + cc * 4, 4)
            pltpu.async_copy(src_hbm.at[pl.ds(b, 4)], si[p], csem[p])

        def wait_chunk(p):
            pltpu.make_async_copy(
                src_hbm.at[pl.ds(0, 4)], si[p], csem[p]).wait()

        def issue_di(i, p):
            pltpu.async_copy(
                dst_hbm.at[pl.ds((e0 // _KH) + i * 2, 2)], di[p], dsem[p])

        def wait_di(p):
            pltpu.make_async_copy(
                dst_hbm.at[pl.ds(0, 2)], di[p], dsem[p]).wait()

        def wait_scatter(h):
            pltpu.make_async_copy(
                g_hbm.at[pl.ds(0, _KH)], rows[h], ssem[h]).wait()

        def wait_gather(h):
            pltpu.make_async_copy(
                g_hbm.at[pl.ds(0, _KH)], rows[h], gsem[h]).wait()

        issue_chunk(0, 0)
        issue_chunk(1, 1)
        issue_di(0, 0)
        issue_di(1, 1)

        def body(u, carry):
            for r in range(8):
                i = u * 8 + r
                sp = (r // 4) % 2

                if r == 0:
                    wait_chunk(0)
                if r == 4:
                    wait_chunk(1)

                    @pl.when(u < (_ITERS // 8) - 1)
                    def _():
                        issue_chunk(u * 2 + 2, 0)

                @pl.when(i >= 1)
                def _():
                    wait_scatter(0)

                wait_di(r % _NSET)

                @pl.when(i + 2 < _ITERS)
                def _():
                    issue_di(i + 2, (r + 2) % _NSET)

                pltpu.async_copy(
                    g_hbm.at[si[sp].at[r % 4, pl.ds(0, _KH)]],
                    rows[0], gsem[0])

                @pl.when(i >= 1)
                def _():
                    wait_scatter(1)

                pltpu.async_copy(
                    g_hbm.at[si[sp].at[r % 4, pl.ds(_KH, _KH)]],
                    rows[1], gsem[1])
                wait_gather(0)
                pltpu.async_copy(rows[0], acc_sh.at[di[r % _NSET].at[0]],
                                 ssem[0], add=True)
                wait_gather(1)
                pltpu.async_copy(rows[1], acc_sh.at[di[r % _NSET].at[1]],
                                 ssem[1], add=True)

            @pl.when(u < (_ITERS // 8) - 1)
            def _():
                issue_chunk(u * 2 + 3, 1)

            return carry

        lax.fori_loop(0, _ITERS // 8, body, 0)
        wait_scatter(0)
        wait_scatter(1)
        plsc.subcore_barrier()
        dump0 = pl.multiple_of(c * _N + row0, 8)
        pltpu.sync_copy(acc_sh.at[pl.ds(row0, _RPS)],
                        out_hbm.at[pl.ds(dump0, _RPS)])

    return k(g, src, dst2h, zrows)


_R = 5248


def _prep1_body(deg_ref, x_ref, w_ref, dinv_ref, g_ref):
    deg = deg_ref[0, :] + deg_ref[1, :] + 1.0
    dinv = lax.rsqrt(deg)
    dinv_ref[...] = dinv[:, None]
    h = jnp.dot(x_ref[...], w_ref[...], preferred_element_type=jnp.float32)
    g_ref[...] = jnp.concatenate(
        [h * dinv[:, None], jnp.zeros((_R, _EMBP - _EMB), jnp.float32)], axis=1)


def _tc_prep1(deg2, x, W1c):
    return pl.pallas_call(
        _prep1_body,
        grid=(_N // _R,),
        in_specs=[
            pl.BlockSpec((2, _R), lambda i: (0, i)),
            pl.BlockSpec((_R, _SEQ), lambda i: (i, 0)),
            pl.BlockSpec((_SEQ, _EMB), lambda i: (0, 0)),
        ],
        out_specs=[
            pl.BlockSpec((_R, 1), lambda i: (i, 0)),
            pl.BlockSpec((_R, _EMBP), lambda i: (i, 0)),
        ],
        out_shape=[
            jax.ShapeDtypeStruct((_N, 1), jnp.float32),
            jax.ShapeDtypeStruct((_N, _EMBP), jnp.float32),
        ],
    )(deg2, x, W1c)


def _mid_body(accp_ref, dinv_ref, b1_ref, w2_ref, g2_ref):
    acc = accp_ref[0, :, :_EMB] + accp_ref[1, :, :_EMB]
    dinv = dinv_ref[...]
    z1 = jnp.maximum(dinv * acc + b1_ref[...], 0.0)
    g2 = jnp.dot(z1, w2_ref[...], preferred_element_type=jnp.float32) * dinv
    g2_ref[...] = jnp.concatenate(
        [g2, jnp.zeros((_R, _EMBP - _EMB), jnp.float32)], axis=1)


def _tc_mid(accp, dinv, b1, W2c):
    return pl.pallas_call(
        _mid_body,
        grid=(_N // _R,),
        in_specs=[
            pl.BlockSpec((2, _R, _EMBP), lambda i: (0, i, 0)),
            pl.BlockSpec((_R, 1), lambda i: (i, 0)),
            pl.BlockSpec((1, _EMB), lambda i: (0, 0)),
            pl.BlockSpec((_EMB, _EMB), lambda i: (0, 0)),
        ],
        out_specs=pl.BlockSpec((_R, _EMBP), lambda i: (i, 0)),
        out_shape=jax.ShapeDtypeStruct((_N, _EMBP), jnp.float32),
    )(accp, dinv, b1, W2c)


def _fin_body(accp_ref, dinv_ref, b2_ref, z2_ref):
    acc = accp_ref[0, :, :_EMB] + accp_ref[1, :, :_EMB]
    z2_ref[...] = jnp.maximum(
        dinv_ref[...] * acc + b2_ref[...], 0.0)


def _tc_fin(accp, dinv, b2):
    return pl.pallas_call(
        _fin_body,
        grid=(_N // _R,),
        in_specs=[
            pl.BlockSpec((2, _R, _EMBP), lambda i: (0, i, 0)),
            pl.BlockSpec((_R, 1), lambda i: (i, 0)),
            pl.BlockSpec((1, _EMB), lambda i: (0, 0)),
        ],
        out_specs=pl.BlockSpec((_R, _EMB), lambda i: (i, 0)),
        out_shape=jax.ShapeDtypeStruct((_N, _EMB), jnp.float32),
    )(accp, dinv, b2)


_GB = 256


def _head_body(lat_ref, wfc_ref, bfc_ref, wout_ref, bout_ref, o_ref):
    h = jnp.maximum(
        jnp.dot(lat_ref[...], wfc_ref[...],
                preferred_element_type=jnp.float32) + bfc_ref[...], 0.0)
    o_ref[...] = jnp.dot(h, wout_ref[...],
                         preferred_element_type=jnp.float32) + bout_ref[...]


def _tc_head(lat, Wfc, bfc, Wout, bout):
    return pl.pallas_call(
        _head_body,
        grid=(_B // _GB,),
        in_specs=[
            pl.BlockSpec((_GB, _NN * _EMB), lambda i: (i, 0)),
            pl.BlockSpec((_NN * _EMB, _HID), lambda i: (0, 0)),
            pl.BlockSpec((1, _HID), lambda i: (0, 0)),
            pl.BlockSpec((_HID, _NCLS), lambda i: (0, 0)),
            pl.BlockSpec((1, _NCLS), lambda i: (0, 0)),
        ],
        out_specs=pl.BlockSpec((_GB, _NCLS), lambda i: (i, 0)),
        out_shape=jax.ShapeDtypeStruct((_B, _NCLS), jnp.float32),
    )(lat, Wfc, bfc, Wout, bout)


def kernel(x, edge_index, batch_index, W1c, b1c, W2c, b2c, Wfc, bfc, Wout, bout):
    src = edge_index[0]
    dst = edge_index[1]
    src4 = src.reshape(_E // _K, _K)
    dst2h = dst.reshape(_E // _KH, _KH)
    dst2 = dst.reshape(_E // _K, _K)
    zrows = jnp.zeros((_N, _EMBP), jnp.float32)

    degp = _sc_degree(dst2)
    deg2 = degp.reshape(2, _N)
    dinv, g1 = _tc_prep1(deg2, x, W1c)

    acc1p = _sc_scatter(g1, src4, dst2h, zrows).reshape(2, _N, _EMBP)
    g2 = _tc_mid(acc1p, dinv, b1c.reshape(1, _EMB), W2c)

    acc2p = _sc_scatter(g2, src4, dst2h, zrows).reshape(2, _N, _EMBP)
    z2 = _tc_fin(acc2p, dinv, b2c.reshape(1, _EMB))

    lat = z2.reshape(_B, _NN * _EMB)
    return _tc_head(lat, Wfc, bfc.reshape(1, _HID), Wout, bout.reshape(1, _NCLS))
